# baseline (speedup 1.0000x reference)
import functools

import jax
import jax.numpy as jnp
from jax import lax
from jax.experimental import pallas as pl
from jax.experimental.pallas import tpu as pltpu
from jax.experimental.pallas import tpu_sc as plsc

_EMBED = 128
_ADIM = 64
_NC, _NS, _L = 2, 16, 16
_NW = _NC * _NS
_CHUNK = 400


@functools.lru_cache(maxsize=None)
def _make_project(n_nodes: int, block: int):
    def body(x_ref, wq_ref, wk_ref, q_ref, k_ref):
        x = x_ref[...]
        ssq = jnp.sum(x * x, axis=1, keepdims=True)
        inv = 1.0 / jnp.maximum(jnp.sqrt(ssq), 1e-12)
        xn = x * inv
        scale = 1.0 / (_ADIM ** 0.5)
        dn = (((1,), (1,)), ((), ()))
        q = lax.dot_general(xn, wq_ref[...], dn,
                            preferred_element_type=jnp.float32) * scale
        k = lax.dot_general(xn, wk_ref[...], dn,
                            preferred_element_type=jnp.float32)
        q_ref[...] = q.astype(jnp.bfloat16)
        k_ref[...] = k.astype(jnp.bfloat16)

    return pl.pallas_call(
        body,
        grid=(n_nodes // block,),
        in_specs=[
            pl.BlockSpec((block, _EMBED), lambda i: (i, 0)),
            pl.BlockSpec((_ADIM, _EMBED), lambda i: (0, 0)),
            pl.BlockSpec((_ADIM, _EMBED), lambda i: (0, 0)),
        ],
        out_specs=[
            pl.BlockSpec((block, _ADIM), lambda i: (i, 0)),
            pl.BlockSpec((block, _ADIM), lambda i: (i, 0)),
        ],
        out_shape=[
            jax.ShapeDtypeStruct((n_nodes, _ADIM), jnp.bfloat16),
            jax.ShapeDtypeStruct((n_nodes, _ADIM), jnp.bfloat16),
        ],
    )


@functools.lru_cache(maxsize=None)
def _make_edge_scores(n_edges: int, n_nodes: int):
    per_w = n_edges // _NW
    n_chunks = per_w // _CHUNK
    assert per_w % _CHUNK == 0 and per_w % 8 == 0

    mesh = plsc.VectorSubcoreMesh(
        core_axis_name="c", subcore_axis_name="s",
        num_cores=_NC, num_subcores=_NS,
    )

    n_sub = 5
    sub = _CHUNK // n_sub

    @functools.partial(
        pl.kernel,
        out_type=jax.ShapeDtypeStruct((n_edges,), jnp.float32),
        mesh=mesh,
        scratch_types=[
            pltpu.VMEM((per_w,), jnp.int32),
            pltpu.VMEM((per_w,), jnp.int32),
            pltpu.VMEM((_CHUNK, _ADIM), jnp.bfloat16),
            pltpu.VMEM((_CHUNK, _ADIM), jnp.bfloat16),
            pltpu.VMEM((_CHUNK, _ADIM), jnp.bfloat16),
            pltpu.VMEM((_CHUNK, _ADIM), jnp.bfloat16),
            pltpu.VMEM((2, _CHUNK), jnp.float32),
            pltpu.SemaphoreType.DMA,
            pltpu.SemaphoreType.DMA,
            pltpu.SemaphoreType.DMA,
            pltpu.SemaphoreType.DMA,
            pltpu.SemaphoreType.DMA,
        ],
        compiler_params=pltpu.CompilerParams(
            needs_layout_passes=False, use_tc_tiling_on_sc=False),
    )
    def edge_scores(q_hbm, k_hbm, ei_hbm, out_hbm,
                    src_all, tgt_all, qr0, qr1, kr0, kr1, sc_v,
                    sem_i, sg0, sg1, so0, so1):
        qr = [qr0, qr1]
        kr = [kr0, kr1]
        sem_g = [sg0, sg1]
        sem_o = [so0, so1]
        wid = lax.axis_index("s") * _NC + lax.axis_index("c")
        w_base = wid * per_w

        def gather_start(c, b):
            for j in range(n_sub):
                off = c * _CHUNK + j * sub
                pltpu.async_copy(q_hbm.at[src_all.at[pl.ds(off, sub)]],
                                 qr[b].at[pl.ds(j * sub, sub), :], sem_g[b])
                pltpu.async_copy(k_hbm.at[tgt_all.at[pl.ds(off, sub)]],
                                 kr[b].at[pl.ds(j * sub, sub), :], sem_g[b])

        def gather_wait(b):
            for j in range(n_sub):
                pltpu.make_async_copy(q_hbm.at[src_all.at[pl.ds(0, sub)]],
                                      qr[b].at[pl.ds(j * sub, sub), :], sem_g[b]).wait()
                pltpu.make_async_copy(k_hbm.at[tgt_all.at[pl.ds(0, sub)]],
                                      kr[b].at[pl.ds(j * sub, sub), :], sem_g[b]).wait()

        def out_start(c, b):
            base = w_base + c * _CHUNK
            pltpu.async_copy(sc_v.at[b], out_hbm.at[pl.ds(base, _CHUNK)], sem_o[b])

        def out_wait(b):
            pltpu.make_async_copy(sc_v.at[b], out_hbm.at[pl.ds(0, _CHUNK)], sem_o[b]).wait()

        def compute(b):
            lane = lax.iota(jnp.int32, _L)
            himask = jnp.full((_L,), -65536, jnp.int32)

            def expand(v):
                iv = plsc.bitcast(v, jnp.int32)
                lo = plsc.bitcast(lax.shift_left(iv, 16), jnp.float32)
                hi = plsc.bitcast(jnp.bitwise_and(iv, himask), jnp.float32)
                return lo, hi

            def group_body(g, carry):
                acc = jnp.zeros((_L,), jnp.float32)
                for e in range(_L):
                    row = g * _L + e
                    p = jnp.zeros((_L,), jnp.float32)
                    for j in range(_ADIM // (2 * _L)):
                        q0, q1 = expand(qr[b][row, pl.ds(j * 2 * _L, 2 * _L)])
                        k0, k1 = expand(kr[b][row, pl.ds(j * 2 * _L, 2 * _L)])
                        p = p + q0 * k0 + q1 * k1
                    s = jnp.sum(p)
                    acc = jnp.where(lane == e, s, acc)
                sc_v[b, pl.ds(g * _L, _L)] = acc
                return carry

            lax.fori_loop(0, _CHUNK // _L, group_body, 0)

        ci_s = pltpu.async_copy(ei_hbm.at[pl.ds(w_base, per_w)], src_all, sem_i)
        ci_t = pltpu.async_copy(ei_hbm.at[pl.ds(n_edges + w_base, per_w)], tgt_all, sem_i)

        ci_s.wait()
        ci_t.wait()
        gather_start(0, 0)

        def pair_body(p, carry):
            for b in (0, 1):
                c = p * 2 + b

                @pl.when(c < n_chunks)
                def _():
                    @pl.when(c + 1 < n_chunks)
                    def _():
                        gather_start(c + 1, 1 - b)

                    gather_wait(b)

                    @pl.when(c >= 2)
                    def _():
                        out_wait(b)

                    compute(b)
                    out_start(c, b)
            return carry

        lax.fori_loop(0, (n_chunks + 1) // 2, pair_body, 0)
        out_wait(0)
        out_wait(1)

    return edge_scores


def kernel(node_embeddings, edge_index, W_q, W_k):
    n_nodes = node_embeddings.shape[0]
    n_edges = edge_index.shape[1]
    project = _make_project(n_nodes, 2000)
    q_tab, k_tab = project(node_embeddings, W_q, W_k)
    edge_scores = _make_edge_scores(n_edges, n_nodes)
    ei = edge_index.astype(jnp.int32).reshape(2 * n_edges)
    return edge_scores(q_tab, k_tab, ei)


# speedup vs baseline: 1.0078x; 1.0078x over previous
import functools

import jax
import jax.numpy as jnp
from jax import lax
from jax.experimental import pallas as pl
from jax.experimental.pallas import tpu as pltpu
from jax.experimental.pallas import tpu_sc as plsc

_EMBED = 128
_ADIM = 64
_NC, _NS, _L = 2, 16, 16
_NW = _NC * _NS
_CHUNK = 400


@functools.lru_cache(maxsize=None)
def _make_project(n_nodes: int, block: int):
    def body(x_ref, wq_ref, wk_ref, q_ref, k_ref):
        x = x_ref[...]
        ssq = jnp.sum(x * x, axis=1, keepdims=True)
        inv = 1.0 / jnp.maximum(jnp.sqrt(ssq), 1e-12)
        xn = x * inv
        scale = 1.0 / (_ADIM ** 0.5)
        dn = (((1,), (1,)), ((), ()))
        q = lax.dot_general(xn, wq_ref[...], dn,
                            preferred_element_type=jnp.float32) * scale
        k = lax.dot_general(xn, wk_ref[...], dn,
                            preferred_element_type=jnp.float32)
        q_ref[...] = q.astype(jnp.bfloat16)
        k_ref[...] = k.astype(jnp.bfloat16)

    return pl.pallas_call(
        body,
        grid=(n_nodes // block,),
        in_specs=[
            pl.BlockSpec((block, _EMBED), lambda i: (i, 0)),
            pl.BlockSpec((_ADIM, _EMBED), lambda i: (0, 0)),
            pl.BlockSpec((_ADIM, _EMBED), lambda i: (0, 0)),
        ],
        out_specs=[
            pl.BlockSpec((block, _ADIM), lambda i: (i, 0)),
            pl.BlockSpec((block, _ADIM), lambda i: (i, 0)),
        ],
        out_shape=[
            jax.ShapeDtypeStruct((n_nodes, _ADIM), jnp.bfloat16),
            jax.ShapeDtypeStruct((n_nodes, _ADIM), jnp.bfloat16),
        ],
    )


@functools.lru_cache(maxsize=None)
def _make_edge_scores(n_edges: int, n_nodes: int):
    per_w = n_edges // _NW
    n_chunks = per_w // _CHUNK
    assert per_w % _CHUNK == 0 and per_w % 8 == 0

    mesh = plsc.VectorSubcoreMesh(
        core_axis_name="c", subcore_axis_name="s",
        num_cores=_NC, num_subcores=_NS,
    )

    n_sub = 5
    sub = _CHUNK // n_sub

    @functools.partial(
        pl.kernel,
        out_type=jax.ShapeDtypeStruct((n_edges,), jnp.float32),
        mesh=mesh,
        scratch_types=[
            pltpu.VMEM_SHARED((n_nodes, _ADIM), jnp.bfloat16),
            pltpu.VMEM_SHARED((n_nodes, _ADIM), jnp.bfloat16),
            pltpu.VMEM((per_w,), jnp.int32),
            pltpu.VMEM((per_w,), jnp.int32),
            pltpu.VMEM((_CHUNK, _ADIM), jnp.bfloat16),
            pltpu.VMEM((_CHUNK, _ADIM), jnp.bfloat16),
            pltpu.VMEM((_CHUNK, _ADIM), jnp.bfloat16),
            pltpu.VMEM((_CHUNK, _ADIM), jnp.bfloat16),
            pltpu.VMEM((2, _CHUNK), jnp.float32),
            pltpu.SemaphoreType.DMA,
            pltpu.SemaphoreType.DMA,
            pltpu.SemaphoreType.DMA,
            pltpu.SemaphoreType.DMA,
            pltpu.SemaphoreType.DMA,
        ],
        compiler_params=pltpu.CompilerParams(
            needs_layout_passes=False, use_tc_tiling_on_sc=False),
    )
    def edge_scores(q_hbm, k_hbm, ei_hbm, out_hbm,
                    q_sh, k_sh, src_all, tgt_all, qr0, qr1, kr0, kr1, sc_v,
                    sem_i, sg0, sg1, so0, so1):
        qr = [qr0, qr1]
        kr = [kr0, kr1]
        sem_g = [sg0, sg1]
        sem_o = [so0, so1]
        wid = lax.axis_index("s") * _NC + lax.axis_index("c")
        w_base = wid * per_w

        def gather_start(c, b):
            for j in range(n_sub):
                off = c * _CHUNK + j * sub
                pltpu.async_copy(q_sh.at[src_all.at[pl.ds(off, sub)]],
                                 qr[b].at[pl.ds(j * sub, sub), :], sem_g[b])
                pltpu.async_copy(k_sh.at[tgt_all.at[pl.ds(off, sub)]],
                                 kr[b].at[pl.ds(j * sub, sub), :], sem_g[b])

        def gather_wait(b):
            for j in range(n_sub):
                pltpu.make_async_copy(q_sh.at[src_all.at[pl.ds(0, sub)]],
                                      qr[b].at[pl.ds(j * sub, sub), :], sem_g[b]).wait()
                pltpu.make_async_copy(k_sh.at[tgt_all.at[pl.ds(0, sub)]],
                                      kr[b].at[pl.ds(j * sub, sub), :], sem_g[b]).wait()

        def out_start(c, b):
            base = w_base + c * _CHUNK
            pltpu.async_copy(sc_v.at[b], out_hbm.at[pl.ds(base, _CHUNK)], sem_o[b])

        def out_wait(b):
            pltpu.make_async_copy(sc_v.at[b], out_hbm.at[pl.ds(0, _CHUNK)], sem_o[b]).wait()

        def compute(b):
            lane = lax.iota(jnp.int32, _L)
            himask = jnp.full((_L,), -65536, jnp.int32)

            def expand(v):
                iv = plsc.bitcast(v, jnp.int32)
                lo = plsc.bitcast(lax.shift_left(iv, 16), jnp.float32)
                hi = plsc.bitcast(jnp.bitwise_and(iv, himask), jnp.float32)
                return lo, hi

            def group_body(g, carry):
                acc = jnp.zeros((_L,), jnp.float32)
                for e in range(_L):
                    row = g * _L + e
                    p = jnp.zeros((_L,), jnp.float32)
                    for j in range(_ADIM // (2 * _L)):
                        q0, q1 = expand(qr[b][row, pl.ds(j * 2 * _L, 2 * _L)])
                        k0, k1 = expand(kr[b][row, pl.ds(j * 2 * _L, 2 * _L)])
                        p = p + q0 * k0 + q1 * k1
                    s = jnp.sum(p)
                    acc = jnp.where(lane == e, s, acc)
                sc_v[b, pl.ds(g * _L, _L)] = acc
                return carry

            lax.fori_loop(0, _CHUNK // _L, group_body, 0)

        ci_s = pltpu.async_copy(ei_hbm.at[pl.ds(w_base, per_w)], src_all, sem_i)
        ci_t = pltpu.async_copy(ei_hbm.at[pl.ds(n_edges + w_base, per_w)], tgt_all, sem_i)

        @pl.when(lax.axis_index("s") == 0)
        def _():
            pltpu.sync_copy(q_hbm, q_sh)
            pltpu.sync_copy(k_hbm, k_sh)

        ci_s.wait()
        ci_t.wait()
        plsc.subcore_barrier()
        gather_start(0, 0)

        def pair_body(p, carry):
            for b in (0, 1):
                c = p * 2 + b

                @pl.when(c < n_chunks)
                def _():
                    @pl.when(c + 1 < n_chunks)
                    def _():
                        gather_start(c + 1, 1 - b)

                    gather_wait(b)

                    @pl.when(c >= 2)
                    def _():
                        out_wait(b)

                    compute(b)
                    out_start(c, b)
            return carry

        lax.fori_loop(0, (n_chunks + 1) // 2, pair_body, 0)
        out_wait(0)
        out_wait(1)

    return edge_scores


def kernel(node_embeddings, edge_index, W_q, W_k):
    n_nodes = node_embeddings.shape[0]
    n_edges = edge_index.shape[1]
    project = _make_project(n_nodes, 2000)
    q_tab, k_tab = project(node_embeddings, W_q, W_k)
    edge_scores = _make_edge_scores(n_edges, n_nodes)
    ei = edge_index.astype(jnp.int32).reshape(2 * n_edges)
    return edge_scores(q_tab, k_tab, ei)
